# Initial kernel scaffold; baseline (speedup 1.0000x reference)
import jax
import jax.numpy as jnp
from jax.experimental import pallas as pl


def kernel(x_piece, edge_index_piece, W1, b1, W2, b2):
    raise NotImplementedError("write your pallas kernel here")


# speedup vs baseline: 17.2804x; 17.2804x over previous
import functools

import jax
import jax.numpy as jnp
from jax import lax
from jax.experimental import pallas as pl
from jax.experimental.pallas import tpu as pltpu
from jax.experimental.pallas import tpu_sc as plsc

N = 100000
E = 6400000
NPAD = 100352
HIST_R = NPAD // 16
NW = 32
EPT = 200704
EPAD = NW * EPT
CH = EPT // 128
CH_D = EPT // 1024
PAD_NODES = NPAD - N

_mesh = plsc.VectorSubcoreMesh(core_axis_name="c", subcore_axis_name="s")
_sc_params = pltpu.CompilerParams(
    needs_layout_passes=False, use_tc_tiling_on_sc=False)


@functools.partial(
    pl.kernel,
    out_type=jax.ShapeDtypeStruct((2, NPAD, 16), jnp.float32),
    mesh=_mesh,
    scratch_types=[
        pltpu.VMEM((HIST_R, 16), jnp.float32),
        pltpu.VMEM((1024,), jnp.int32),
        pltpu.VMEM((128,), jnp.int32),
        pltpu.VMEM((49, 16), jnp.float32),
        pltpu.VMEM((784, 16), jnp.float32),
        pltpu.VMEM_SHARED((HIST_R, 16), jnp.float32),
    ],
    compiler_params=_sc_params,
)
def _deg_kernel(dst_hbm, out_hbm, hist, idxb, ridx, wb_in, wb_out, acc):
    c = lax.axis_index("c")
    s = lax.axis_index("s")
    w = c * 16 + s
    zeros16 = jnp.zeros((16,), jnp.float32)
    ones16 = jnp.ones((16,), jnp.float32)
    iota16 = lax.iota(jnp.int32, 16)

    def _z(i, _):
        hist[i] = zeros16
        return 0
    lax.fori_loop(0, HIST_R, _z, 0)

    def _z2(i, _):
        wb_out[i] = zeros16
        return 0
    lax.fori_loop(0, 784, _z2, 0)
    R_T = HIST_R // 16
    pltpu.sync_copy(wb_out.at[pl.ds(0, R_T)], acc.at[pl.ds(s * R_T, R_T)])
    plsc.subcore_barrier()

    def _count(k, _):
        pltpu.sync_copy(dst_hbm.at[w, pl.ds(k * 1024, 1024)], idxb)
        for j in range(64):
            d = idxb[pl.ds(j * 16, 16)]
            r = jnp.right_shift(d, 4)
            l = jnp.bitwise_and(d, 15)
            plsc.addupdate_scatter(hist, [r, l], ones16)
        return 0
    lax.fori_loop(0, CH_D, _count, 0)

    def _red(rc, _):
        base = rc * 128
        for j in range(8):
            ridx[pl.ds(j * 16, 16)] = iota16 + (base + j * 16)
        pltpu.sync_copy(hist.at[pl.ds(base, 128)], acc.at[ridx], add=True)
        return 0
    lax.fori_loop(0, HIST_R // 128, _red, 0)
    plsc.subcore_barrier()

    def _wb(k, _):
        r0 = s * R_T + k * 49
        pltpu.sync_copy(acc.at[pl.ds(r0, 49)], wb_in)

        def _rep(j, _):
            rr = jnp.full((16,), jnp.right_shift(j, 4), jnp.int32)
            ll = jnp.full((16,), jnp.bitwise_and(j, 15), jnp.int32)
            wb_out[j] = plsc.load_gather(wb_in, [rr, ll])
            return 0
        lax.fori_loop(0, 784, _rep, 0)
        pltpu.sync_copy(wb_out, out_hbm.at[c, pl.ds(s * 6272 + k * 784, 784)])
        return 0
    lax.fori_loop(0, 8, _wb, 0)


@functools.partial(
    pl.kernel,
    out_type=jax.ShapeDtypeStruct((2, NPAD, 16), jnp.float32),
    mesh=_mesh,
    scratch_types=[
        pltpu.VMEM((128,), jnp.int32),
        pltpu.VMEM((128,), jnp.int32),
        pltpu.VMEM((128, 16), jnp.float32),
        pltpu.VMEM((784, 16), jnp.float32),
        pltpu.VMEM_SHARED((NPAD, 16), jnp.float32),
        pltpu.SemaphoreType.DMA,
    ],
    compiler_params=_sc_params,
)
def _pass_kernel(g_hbm, src_hbm, dst_hbm, out_hbm, idx_s, idx_d, rows, wb,
                 acc, sem):
    c = lax.axis_index("c")
    s = lax.axis_index("s")
    w = c * 16 + s
    zeros16 = jnp.zeros((16,), jnp.float32)

    def _z(i, _):
        wb[i] = zeros16
        return 0
    lax.fori_loop(0, 784, _z, 0)
    base_rows = s * (NPAD // 16)

    def _zc(k, _):
        pltpu.sync_copy(wb, acc.at[pl.ds(base_rows + k * 784, 784)])
        return 0
    lax.fori_loop(0, 8, _zc, 0)
    plsc.subcore_barrier()

    def _edge(k, _):
        pltpu.sync_copy(src_hbm.at[w, pl.ds(k * 128, 128)], idx_s)
        pltpu.sync_copy(dst_hbm.at[w, pl.ds(k * 128, 128)], idx_d)
        pltpu.async_copy(g_hbm.at[idx_s], rows, sem).wait()
        pltpu.sync_copy(rows, acc.at[idx_d], add=True)
        return 0
    lax.fori_loop(0, CH, _edge, 0)
    plsc.subcore_barrier()

    def _wb(k, _):
        r0 = base_rows + k * 784
        pltpu.sync_copy(acc.at[pl.ds(r0, 784)], wb)
        pltpu.sync_copy(wb, out_hbm.at[c, pl.ds(r0, 784)])
        return 0
    lax.fori_loop(0, 8, _wb, 0)


BLK = 6272
GRID = NPAD // BLK


def _tcA_body(x_ref, w1_ref, degp_ref, g1_ref, dis_ref):
    deg = degp_ref[0] + degp_ref[1] + 1.0
    dis = lax.rsqrt(deg)
    h = jnp.dot(x_ref[...], w1_ref[...], preferred_element_type=jnp.float32)
    g1_ref[...] = h * dis
    dis_ref[...] = dis


def _tcA(x_pad, W1, degp):
    return pl.pallas_call(
        _tcA_body,
        grid=(GRID,),
        in_specs=[
            pl.BlockSpec((BLK, 8), lambda i: (i, 0)),
            pl.BlockSpec((8, 16), lambda i: (0, 0)),
            pl.BlockSpec((2, BLK, 16), lambda i: (0, i, 0)),
        ],
        out_specs=[
            pl.BlockSpec((BLK, 16), lambda i: (i, 0)),
            pl.BlockSpec((BLK, 16), lambda i: (i, 0)),
        ],
        out_shape=[
            jax.ShapeDtypeStruct((NPAD, 16), jnp.float32),
            jax.ShapeDtypeStruct((NPAD, 16), jnp.float32),
        ],
    )(x_pad, W1, degp)


def _tcB_body(s1_ref, g1_ref, dis_ref, b1_ref, w2_ref, g2a_ref, g2b_ref):
    dis = dis_ref[...]
    o1 = jnp.maximum(dis * (s1_ref[0] + s1_ref[1] + g1_ref[...]) + b1_ref[...],
                     0.0)
    h2 = jnp.dot(o1, w2_ref[...], preferred_element_type=jnp.float32)
    g2a_ref[...] = h2[:, :16] * dis
    g2b_ref[...] = h2[:, 16:] * dis


def _tcB(S1, g1, dis, b1, W2):
    return pl.pallas_call(
        _tcB_body,
        grid=(GRID,),
        in_specs=[
            pl.BlockSpec((2, BLK, 16), lambda i: (0, i, 0)),
            pl.BlockSpec((BLK, 16), lambda i: (i, 0)),
            pl.BlockSpec((BLK, 16), lambda i: (i, 0)),
            pl.BlockSpec((1, 16), lambda i: (0, 0)),
            pl.BlockSpec((16, 32), lambda i: (0, 0)),
        ],
        out_specs=[
            pl.BlockSpec((BLK, 16), lambda i: (i, 0)),
            pl.BlockSpec((BLK, 16), lambda i: (i, 0)),
        ],
        out_shape=[
            jax.ShapeDtypeStruct((NPAD, 16), jnp.float32),
            jax.ShapeDtypeStruct((NPAD, 16), jnp.float32),
        ],
    )(S1, g1, dis, b1, W2)


def _tcC_body(s2a_ref, s2b_ref, g2a_ref, g2b_ref, dis_ref, b2_ref, out_ref):
    dis = dis_ref[...]
    oa = dis * (s2a_ref[0] + s2a_ref[1] + g2a_ref[...]) + b2_ref[:, :16]
    ob = dis * (s2b_ref[0] + s2b_ref[1] + g2b_ref[...]) + b2_ref[:, 16:]
    out_ref[...] = jnp.concatenate([oa, ob], axis=1)


def _tcC(S2a, S2b, g2a, g2b, dis, b2):
    return pl.pallas_call(
        _tcC_body,
        grid=(GRID,),
        in_specs=[
            pl.BlockSpec((2, BLK, 16), lambda i: (0, i, 0)),
            pl.BlockSpec((2, BLK, 16), lambda i: (0, i, 0)),
            pl.BlockSpec((BLK, 16), lambda i: (i, 0)),
            pl.BlockSpec((BLK, 16), lambda i: (i, 0)),
            pl.BlockSpec((BLK, 16), lambda i: (i, 0)),
            pl.BlockSpec((1, 32), lambda i: (0, 0)),
        ],
        out_specs=pl.BlockSpec((BLK, 32), lambda i: (i, 0)),
        out_shape=jax.ShapeDtypeStruct((NPAD, 32), jnp.float32),
    )(S2a, S2b, g2a, g2b, dis, b2)


def kernel(x_piece, edge_index_piece, W1, b1, W2, b2):
    x_pad = jnp.pad(x_piece, ((0, NPAD - N), (0, 0)))
    n_fake = EPAD - E
    fake = (N + (jnp.arange(n_fake, dtype=jnp.int32) % PAD_NODES))
    src2 = jnp.concatenate([edge_index_piece[0], fake]).reshape(NW, EPT)
    dst2 = jnp.concatenate([edge_index_piece[1], fake]).reshape(NW, EPT)

    degp = _deg_kernel(dst2)
    g1, dis = _tcA(x_pad, W1, degp)
    S1 = _pass_kernel(g1, src2, dst2)
    g2a, g2b = _tcB(S1, g1, dis, b1.reshape(1, 16), W2)
    S2a = _pass_kernel(g2a, src2, dst2)
    S2b = _pass_kernel(g2b, src2, dst2)
    out = _tcC(S2a, S2b, g2a, g2b, dis, b2.reshape(1, 32))
    return out[:N]


# speedup vs baseline: 57.5489x; 3.3303x over previous
import functools

import jax
import jax.numpy as jnp
from jax import lax
from jax.experimental import pallas as pl
from jax.experimental.pallas import tpu as pltpu
from jax.experimental.pallas import tpu_sc as plsc

N = 100000
E = 6400000
NPAD = 100352
HIST_R = NPAD // 16
NW = 32
EPT = 200704
EPAD = NW * EPT
CH = EPT // 128
CH_D = EPT // 1024
PAD_NODES = NPAD - N

_mesh = plsc.VectorSubcoreMesh(core_axis_name="c", subcore_axis_name="s")
_sc_params = pltpu.CompilerParams(
    needs_layout_passes=False, use_tc_tiling_on_sc=False)


@functools.partial(
    pl.kernel,
    out_type=jax.ShapeDtypeStruct((2, NPAD, 16), jnp.float32),
    mesh=_mesh,
    scratch_types=[
        pltpu.VMEM((HIST_R, 16), jnp.float32),
        pltpu.VMEM((1024,), jnp.int32),
        pltpu.VMEM((128,), jnp.int32),
        pltpu.VMEM((49, 16), jnp.float32),
        pltpu.VMEM((784, 16), jnp.float32),
        pltpu.VMEM_SHARED((HIST_R, 16), jnp.float32),
    ],
    compiler_params=_sc_params,
)
def _deg_kernel(dst_hbm, out_hbm, hist, idxb, ridx, wb_in, wb_out, acc):
    c = lax.axis_index("c")
    s = lax.axis_index("s")
    w = c * 16 + s
    zeros16 = jnp.zeros((16,), jnp.float32)
    ones16 = jnp.ones((16,), jnp.float32)
    iota16 = lax.iota(jnp.int32, 16)

    def _z(i, _):
        hist[i] = zeros16
        return 0
    lax.fori_loop(0, HIST_R, _z, 0)

    def _z2(i, _):
        wb_out[i] = zeros16
        return 0
    lax.fori_loop(0, 784, _z2, 0)
    R_T = HIST_R // 16
    pltpu.sync_copy(wb_out.at[pl.ds(0, R_T)], acc.at[pl.ds(s * R_T, R_T)])
    plsc.subcore_barrier()

    def _count(k, _):
        pltpu.sync_copy(dst_hbm.at[w, pl.ds(k * 1024, 1024)], idxb)
        for j in range(64):
            d = idxb[pl.ds(j * 16, 16)]
            r = jnp.right_shift(d, 4)
            l = jnp.bitwise_and(d, 15)
            plsc.addupdate_scatter(hist, [r, l], ones16)
        return 0
    lax.fori_loop(0, CH_D, _count, 0)

    def _red(rc, _):
        base = rc * 128
        for j in range(8):
            ridx[pl.ds(j * 16, 16)] = iota16 + (base + j * 16)
        pltpu.sync_copy(hist.at[pl.ds(base, 128)], acc.at[ridx], add=True)
        return 0
    lax.fori_loop(0, HIST_R // 128, _red, 0)
    plsc.subcore_barrier()

    def _wb(k, _):
        r0 = s * R_T + k * 49
        pltpu.sync_copy(acc.at[pl.ds(r0, 49)], wb_in)

        def _rep(j, _):
            rr = jnp.full((16,), jnp.right_shift(j, 4), jnp.int32)
            ll = jnp.full((16,), jnp.bitwise_and(j, 15), jnp.int32)
            wb_out[j] = plsc.load_gather(wb_in, [rr, ll])
            return 0
        lax.fori_loop(0, 784, _rep, 0)
        pltpu.sync_copy(wb_out, out_hbm.at[c, pl.ds(s * 6272 + k * 784, 784)])
        return 0
    lax.fori_loop(0, 8, _wb, 0)


@functools.partial(
    pl.kernel,
    out_type=jax.ShapeDtypeStruct((2, NPAD, 16), jnp.float32),
    mesh=_mesh,
    scratch_types=[
        pltpu.VMEM((8, 128), jnp.int32),
        pltpu.VMEM((8, 128), jnp.int32),
        pltpu.VMEM((8, 128, 16), jnp.float32),
        pltpu.VMEM((392, 16), jnp.float32),
        pltpu.VMEM_SHARED((NPAD, 16), jnp.float32),
        pltpu.SemaphoreType.DMA,
        pltpu.SemaphoreType.DMA,
        pltpu.SemaphoreType.DMA,
    ],
    compiler_params=_sc_params,
)
def _pass_kernel(g_hbm, src_hbm, dst_hbm, out_hbm, idx_s, idx_d, rows, wb,
                 acc, sem_i, sem_g, sem_sc):
    c = lax.axis_index("c")
    s = lax.axis_index("s")
    w = c * 16 + s
    zeros16 = jnp.zeros((16,), jnp.float32)

    def _z(i, _):
        wb[i] = zeros16
        return 0
    lax.fori_loop(0, 392, _z, 0)
    base_rows = s * (NPAD // 16)

    def _zc(k, _):
        pltpu.sync_copy(wb, acc.at[pl.ds(base_rows + k * 392, 392)])
        return 0
    lax.fori_loop(0, 16, _zc, 0)
    plsc.subcore_barrier()

    def _edge(g, _):
        k0 = g * 8
        hi = [pltpu.async_copy(src_hbm.at[w, pl.ds((k0 + b) * 128, 128)],
                               idx_s.at[b], sem_i) for b in range(8)]
        hi += [pltpu.async_copy(dst_hbm.at[w, pl.ds((k0 + b) * 128, 128)],
                                idx_d.at[b], sem_i) for b in range(8)]
        for h in hi:
            h.wait()
        hg = [pltpu.async_copy(g_hbm.at[idx_s.at[b]], rows.at[b], sem_g)
              for b in range(8)]
        for h in hg:
            h.wait()
        hs = [pltpu.async_copy(rows.at[b], acc.at[idx_d.at[b]], sem_sc,
                               add=True) for b in range(8)]
        for h in hs:
            h.wait()
        return 0
    lax.fori_loop(0, CH // 8, _edge, 0)
    plsc.subcore_barrier()

    def _wb(k, _):
        r0 = base_rows + k * 392
        pltpu.sync_copy(acc.at[pl.ds(r0, 392)], wb)
        pltpu.sync_copy(wb, out_hbm.at[c, pl.ds(r0, 392)])
        return 0
    lax.fori_loop(0, 16, _wb, 0)


BLK = 6272
GRID = NPAD // BLK


def _tcA_body(x_ref, w1_ref, degp_ref, g1_ref, dis_ref):
    deg = degp_ref[0] + degp_ref[1] + 1.0
    dis = lax.rsqrt(deg)
    h = jnp.dot(x_ref[...], w1_ref[...], preferred_element_type=jnp.float32)
    g1_ref[...] = h * dis
    dis_ref[...] = dis


def _tcA(x_pad, W1, degp):
    return pl.pallas_call(
        _tcA_body,
        grid=(GRID,),
        in_specs=[
            pl.BlockSpec((BLK, 8), lambda i: (i, 0)),
            pl.BlockSpec((8, 16), lambda i: (0, 0)),
            pl.BlockSpec((2, BLK, 16), lambda i: (0, i, 0)),
        ],
        out_specs=[
            pl.BlockSpec((BLK, 16), lambda i: (i, 0)),
            pl.BlockSpec((BLK, 16), lambda i: (i, 0)),
        ],
        out_shape=[
            jax.ShapeDtypeStruct((NPAD, 16), jnp.float32),
            jax.ShapeDtypeStruct((NPAD, 16), jnp.float32),
        ],
    )(x_pad, W1, degp)


def _tcB_body(s1_ref, g1_ref, dis_ref, b1_ref, w2_ref, g2a_ref, g2b_ref):
    dis = dis_ref[...]
    o1 = jnp.maximum(dis * (s1_ref[0] + s1_ref[1] + g1_ref[...]) + b1_ref[...],
                     0.0)
    h2 = jnp.dot(o1, w2_ref[...], preferred_element_type=jnp.float32)
    g2a_ref[...] = h2[:, :16] * dis
    g2b_ref[...] = h2[:, 16:] * dis


def _tcB(S1, g1, dis, b1, W2):
    return pl.pallas_call(
        _tcB_body,
        grid=(GRID,),
        in_specs=[
            pl.BlockSpec((2, BLK, 16), lambda i: (0, i, 0)),
            pl.BlockSpec((BLK, 16), lambda i: (i, 0)),
            pl.BlockSpec((BLK, 16), lambda i: (i, 0)),
            pl.BlockSpec((1, 16), lambda i: (0, 0)),
            pl.BlockSpec((16, 32), lambda i: (0, 0)),
        ],
        out_specs=[
            pl.BlockSpec((BLK, 16), lambda i: (i, 0)),
            pl.BlockSpec((BLK, 16), lambda i: (i, 0)),
        ],
        out_shape=[
            jax.ShapeDtypeStruct((NPAD, 16), jnp.float32),
            jax.ShapeDtypeStruct((NPAD, 16), jnp.float32),
        ],
    )(S1, g1, dis, b1, W2)


def _tcC_body(s2a_ref, s2b_ref, g2a_ref, g2b_ref, dis_ref, b2_ref, out_ref):
    dis = dis_ref[...]
    oa = dis * (s2a_ref[0] + s2a_ref[1] + g2a_ref[...]) + b2_ref[:, :16]
    ob = dis * (s2b_ref[0] + s2b_ref[1] + g2b_ref[...]) + b2_ref[:, 16:]
    out_ref[...] = jnp.concatenate([oa, ob], axis=1)


def _tcC(S2a, S2b, g2a, g2b, dis, b2):
    return pl.pallas_call(
        _tcC_body,
        grid=(GRID,),
        in_specs=[
            pl.BlockSpec((2, BLK, 16), lambda i: (0, i, 0)),
            pl.BlockSpec((2, BLK, 16), lambda i: (0, i, 0)),
            pl.BlockSpec((BLK, 16), lambda i: (i, 0)),
            pl.BlockSpec((BLK, 16), lambda i: (i, 0)),
            pl.BlockSpec((BLK, 16), lambda i: (i, 0)),
            pl.BlockSpec((1, 32), lambda i: (0, 0)),
        ],
        out_specs=pl.BlockSpec((BLK, 32), lambda i: (i, 0)),
        out_shape=jax.ShapeDtypeStruct((NPAD, 32), jnp.float32),
    )(S2a, S2b, g2a, g2b, dis, b2)


def kernel(x_piece, edge_index_piece, W1, b1, W2, b2):
    x_pad = jnp.pad(x_piece, ((0, NPAD - N), (0, 0)))
    n_fake = EPAD - E
    fake = (N + (jnp.arange(n_fake, dtype=jnp.int32) % PAD_NODES))
    src2 = jnp.concatenate([edge_index_piece[0], fake]).reshape(NW, EPT)
    dst2 = jnp.concatenate([edge_index_piece[1], fake]).reshape(NW, EPT)

    degp = _deg_kernel(dst2)
    g1, dis = _tcA(x_pad, W1, degp)
    S1 = _pass_kernel(g1, src2, dst2)
    g2a, g2b = _tcB(S1, g1, dis, b1.reshape(1, 16), W2)
    S2a = _pass_kernel(g2a, src2, dst2)
    S2b = _pass_kernel(g2b, src2, dst2)
    out = _tcC(S2a, S2b, g2a, g2b, dis, b2.reshape(1, 32))
    return out[:N]


# speedup vs baseline: 65.3394x; 1.1354x over previous
import functools

import jax
import jax.numpy as jnp
from jax import lax
from jax.experimental import pallas as pl
from jax.experimental.pallas import tpu as pltpu
from jax.experimental.pallas import tpu_sc as plsc

N = 100000
E = 6400000
NPAD = 100352
HIST_R = NPAD // 16
NW = 32
EPT = 200704
EPAD = NW * EPT
CH = EPT // 128
CH_D = EPT // 1024
PAD_NODES = NPAD - N

_mesh = plsc.VectorSubcoreMesh(core_axis_name="c", subcore_axis_name="s")
_sc_params = pltpu.CompilerParams(
    needs_layout_passes=False, use_tc_tiling_on_sc=False)


@functools.partial(
    pl.kernel,
    out_type=jax.ShapeDtypeStruct((2, NPAD, 16), jnp.float32),
    mesh=_mesh,
    scratch_types=[
        pltpu.VMEM((HIST_R, 16), jnp.float32),
        pltpu.VMEM((1024,), jnp.int32),
        pltpu.VMEM((128,), jnp.int32),
        pltpu.VMEM((49, 16), jnp.float32),
        pltpu.VMEM((784, 16), jnp.float32),
        pltpu.VMEM_SHARED((HIST_R, 16), jnp.float32),
    ],
    compiler_params=_sc_params,
)
def _deg_kernel(dst_hbm, out_hbm, hist, idxb, ridx, wb_in, wb_out, acc):
    c = lax.axis_index("c")
    s = lax.axis_index("s")
    w = c * 16 + s
    zeros16 = jnp.zeros((16,), jnp.float32)
    ones16 = jnp.ones((16,), jnp.float32)
    iota16 = lax.iota(jnp.int32, 16)

    def _z(i, _):
        hist[i] = zeros16
        return 0
    lax.fori_loop(0, HIST_R, _z, 0)

    def _z2(i, _):
        wb_out[i] = zeros16
        return 0
    lax.fori_loop(0, 784, _z2, 0)
    R_T = HIST_R // 16
    pltpu.sync_copy(wb_out.at[pl.ds(0, R_T)], acc.at[pl.ds(s * R_T, R_T)])
    plsc.subcore_barrier()

    def _count(k, _):
        pltpu.sync_copy(dst_hbm.at[w, pl.ds(k * 1024, 1024)], idxb)
        for j in range(64):
            d = idxb[pl.ds(j * 16, 16)]
            r = jnp.right_shift(d, 4)
            l = jnp.bitwise_and(d, 15)
            plsc.addupdate_scatter(hist, [r, l], ones16)
        return 0
    lax.fori_loop(0, CH_D, _count, 0)

    def _red(rc, _):
        base = rc * 128
        for j in range(8):
            ridx[pl.ds(j * 16, 16)] = iota16 + (base + j * 16)
        pltpu.sync_copy(hist.at[pl.ds(base, 128)], acc.at[ridx], add=True)
        return 0
    lax.fori_loop(0, HIST_R // 128, _red, 0)
    plsc.subcore_barrier()

    def _wb(k, _):
        r0 = s * R_T + k * 49
        pltpu.sync_copy(acc.at[pl.ds(r0, 49)], wb_in)

        def _rep(j, _):
            rr = jnp.full((16,), jnp.right_shift(j, 4), jnp.int32)
            ll = jnp.full((16,), jnp.bitwise_and(j, 15), jnp.int32)
            wb_out[j] = plsc.load_gather(wb_in, [rr, ll])
            return 0
        lax.fori_loop(0, 784, _rep, 0)
        pltpu.sync_copy(wb_out, out_hbm.at[c, pl.ds(s * 6272 + k * 784, 784)])
        return 0
    lax.fori_loop(0, 8, _wb, 0)


@functools.partial(
    pl.kernel,
    out_type=jax.ShapeDtypeStruct((2, NPAD, 16), jnp.float32),
    mesh=_mesh,
    scratch_types=[
        pltpu.VMEM((4, 4, 128), jnp.int32),
        pltpu.VMEM((4, 4, 128), jnp.int32),
        pltpu.VMEM((2, 4, 128, 16), jnp.float32),
        pltpu.VMEM((392, 16), jnp.float32),
        pltpu.VMEM_SHARED((NPAD, 16), jnp.float32),
        pltpu.SemaphoreType.DMA,
        pltpu.SemaphoreType.DMA,
        pltpu.SemaphoreType.DMA,
    ],
    compiler_params=_sc_params,
)
def _pass_kernel(g_hbm, src_hbm, dst_hbm, out_hbm, idx_s, idx_d, rows, wb,
                 acc, sem_i, sem_g, sem_sc):
    c = lax.axis_index("c")
    s = lax.axis_index("s")
    w = c * 16 + s
    zeros16 = jnp.zeros((16,), jnp.float32)
    NG = CH // 4

    def _z(i, _):
        wb[i] = zeros16
        return 0
    lax.fori_loop(0, 392, _z, 0)
    base_rows = s * (NPAD // 16)

    def _zc(k, _):
        pltpu.sync_copy(wb, acc.at[pl.ds(base_rows + k * 392, 392)])
        return 0
    lax.fori_loop(0, 16, _zc, 0)
    plsc.subcore_barrier()

    def _issue_idx(g):
        r = lax.rem(g, 4)
        for b in range(4):
            pltpu.async_copy(src_hbm.at[w, pl.ds((g * 4 + b) * 128, 128)],
                             idx_s.at[r, b], sem_i)
            pltpu.async_copy(dst_hbm.at[w, pl.ds((g * 4 + b) * 128, 128)],
                             idx_d.at[r, b], sem_i)

    def _drain_idx(g):
        r = lax.rem(g, 4)
        for b in range(4):
            pltpu.make_async_copy(src_hbm.at[w, pl.ds(0, 128)],
                                  idx_s.at[r, b], sem_i).wait()
            pltpu.make_async_copy(dst_hbm.at[w, pl.ds(0, 128)],
                                  idx_d.at[r, b], sem_i).wait()

    def _issue_gather(g):
        r = lax.rem(g, 4)
        p = lax.rem(g, 2)
        for b in range(4):
            pltpu.async_copy(g_hbm.at[idx_s.at[r, b]], rows.at[p, b], sem_g)

    def _drain_gather(g):
        r = lax.rem(g, 4)
        p = lax.rem(g, 2)
        for b in range(4):
            pltpu.make_async_copy(g_hbm.at[idx_s.at[r, b]], rows.at[p, b],
                                  sem_g).wait()

    def _issue_scatter(g):
        r = lax.rem(g, 4)
        p = lax.rem(g, 2)
        for b in range(4):
            pltpu.async_copy(rows.at[p, b], acc.at[idx_d.at[r, b]], sem_sc,
                             add=True)

    def _drain_scatter(g):
        r = lax.rem(g, 4)
        p = lax.rem(g, 2)
        for b in range(4):
            pltpu.make_async_copy(rows.at[p, b], acc.at[idx_d.at[r, b]],
                                  sem_sc).wait()

    _issue_idx(jnp.int32(0))

    def _pipe(i, _):
        @pl.when(jnp.logical_and(i >= 2, i <= NG + 1))
        def _():
            _drain_scatter(i - 2)

        @pl.when(i <= NG - 1)
        def _():
            _drain_idx(i)

        @pl.when(jnp.logical_and(i >= 1, i <= NG))
        def _():
            _drain_gather(i - 1)
            _issue_scatter(i - 1)

        @pl.when(i <= NG - 1)
        def _():
            _issue_gather(i)

        @pl.when(i <= NG - 2)
        def _():
            _issue_idx(i + 1)
        return 0
    lax.fori_loop(0, NG + 2, _pipe, 0)
    plsc.subcore_barrier()

    def _wb(k, _):
        r0 = base_rows + k * 392
        pltpu.sync_copy(acc.at[pl.ds(r0, 392)], wb)
        pltpu.sync_copy(wb, out_hbm.at[c, pl.ds(r0, 392)])
        return 0
    lax.fori_loop(0, 16, _wb, 0)


BLK = 6272
GRID = NPAD // BLK


def _tcA_body(x_ref, w1_ref, degp_ref, g1_ref, dis_ref):
    deg = degp_ref[0] + degp_ref[1] + 1.0
    dis = lax.rsqrt(deg)
    h = jnp.dot(x_ref[...], w1_ref[...], preferred_element_type=jnp.float32)
    g1_ref[...] = h * dis
    dis_ref[...] = dis


def _tcA(x_pad, W1, degp):
    return pl.pallas_call(
        _tcA_body,
        grid=(GRID,),
        in_specs=[
            pl.BlockSpec((BLK, 8), lambda i: (i, 0)),
            pl.BlockSpec((8, 16), lambda i: (0, 0)),
            pl.BlockSpec((2, BLK, 16), lambda i: (0, i, 0)),
        ],
        out_specs=[
            pl.BlockSpec((BLK, 16), lambda i: (i, 0)),
            pl.BlockSpec((BLK, 16), lambda i: (i, 0)),
        ],
        out_shape=[
            jax.ShapeDtypeStruct((NPAD, 16), jnp.float32),
            jax.ShapeDtypeStruct((NPAD, 16), jnp.float32),
        ],
    )(x_pad, W1, degp)


def _tcB_body(s1_ref, g1_ref, dis_ref, b1_ref, w2_ref, g2a_ref, g2b_ref):
    dis = dis_ref[...]
    o1 = jnp.maximum(dis * (s1_ref[0] + s1_ref[1] + g1_ref[...]) + b1_ref[...],
                     0.0)
    h2 = jnp.dot(o1, w2_ref[...], preferred_element_type=jnp.float32)
    g2a_ref[...] = h2[:, :16] * dis
    g2b_ref[...] = h2[:, 16:] * dis


def _tcB(S1, g1, dis, b1, W2):
    return pl.pallas_call(
        _tcB_body,
        grid=(GRID,),
        in_specs=[
            pl.BlockSpec((2, BLK, 16), lambda i: (0, i, 0)),
            pl.BlockSpec((BLK, 16), lambda i: (i, 0)),
            pl.BlockSpec((BLK, 16), lambda i: (i, 0)),
            pl.BlockSpec((1, 16), lambda i: (0, 0)),
            pl.BlockSpec((16, 32), lambda i: (0, 0)),
        ],
        out_specs=[
            pl.BlockSpec((BLK, 16), lambda i: (i, 0)),
            pl.BlockSpec((BLK, 16), lambda i: (i, 0)),
        ],
        out_shape=[
            jax.ShapeDtypeStruct((NPAD, 16), jnp.float32),
            jax.ShapeDtypeStruct((NPAD, 16), jnp.float32),
        ],
    )(S1, g1, dis, b1, W2)


def _tcC_body(s2a_ref, s2b_ref, g2a_ref, g2b_ref, dis_ref, b2_ref, out_ref):
    dis = dis_ref[...]
    oa = dis * (s2a_ref[0] + s2a_ref[1] + g2a_ref[...]) + b2_ref[:, :16]
    ob = dis * (s2b_ref[0] + s2b_ref[1] + g2b_ref[...]) + b2_ref[:, 16:]
    out_ref[...] = jnp.concatenate([oa, ob], axis=1)


def _tcC(S2a, S2b, g2a, g2b, dis, b2):
    return pl.pallas_call(
        _tcC_body,
        grid=(GRID,),
        in_specs=[
            pl.BlockSpec((2, BLK, 16), lambda i: (0, i, 0)),
            pl.BlockSpec((2, BLK, 16), lambda i: (0, i, 0)),
            pl.BlockSpec((BLK, 16), lambda i: (i, 0)),
            pl.BlockSpec((BLK, 16), lambda i: (i, 0)),
            pl.BlockSpec((BLK, 16), lambda i: (i, 0)),
            pl.BlockSpec((1, 32), lambda i: (0, 0)),
        ],
        out_specs=pl.BlockSpec((BLK, 32), lambda i: (i, 0)),
        out_shape=jax.ShapeDtypeStruct((NPAD, 32), jnp.float32),
    )(S2a, S2b, g2a, g2b, dis, b2)


def kernel(x_piece, edge_index_piece, W1, b1, W2, b2):
    x_pad = jnp.pad(x_piece, ((0, NPAD - N), (0, 0)))
    n_fake = EPAD - E
    fake = (N + (jnp.arange(n_fake, dtype=jnp.int32) % PAD_NODES))
    src2 = jnp.concatenate([edge_index_piece[0], fake]).reshape(NW, EPT)
    dst2 = jnp.concatenate([edge_index_piece[1], fake]).reshape(NW, EPT)

    degp = _deg_kernel(dst2)
    g1, dis = _tcA(x_pad, W1, degp)
    S1 = _pass_kernel(g1, src2, dst2)
    g2a, g2b = _tcB(S1, g1, dis, b1.reshape(1, 16), W2)
    S2a = _pass_kernel(g2a, src2, dst2)
    S2b = _pass_kernel(g2b, src2, dst2)
    out = _tcC(S2a, S2b, g2a, g2b, dis, b2.reshape(1, 32))
    return out[:N]


# speedup vs baseline: 68.3666x; 1.0463x over previous
import functools

import jax
import jax.numpy as jnp
from jax import lax
from jax.experimental import pallas as pl
from jax.experimental.pallas import tpu as pltpu
from jax.experimental.pallas import tpu_sc as plsc

N = 100000
E = 6400000
NPAD = 100352
HIST_R = NPAD // 16
NW = 32
EPT = 200704
EPAD = NW * EPT
CH = EPT // 128
CH_D = EPT // 1024
PAD_NODES = NPAD - N

_mesh = plsc.VectorSubcoreMesh(core_axis_name="c", subcore_axis_name="s")
_sc_params = pltpu.CompilerParams(
    needs_layout_passes=False, use_tc_tiling_on_sc=False)


@functools.partial(
    pl.kernel,
    out_type=jax.ShapeDtypeStruct((2, NPAD, 16), jnp.float32),
    mesh=_mesh,
    scratch_types=[
        pltpu.VMEM((HIST_R, 16), jnp.float32),
        pltpu.VMEM((2, 1024), jnp.int32),
        pltpu.VMEM((128,), jnp.int32),
        pltpu.VMEM((49, 16), jnp.float32),
        pltpu.VMEM((784, 16), jnp.float32),
        pltpu.VMEM_SHARED((HIST_R, 16), jnp.float32),
        pltpu.SemaphoreType.DMA,
    ],
    compiler_params=_sc_params,
)
def _deg_kernel(dst_hbm, out_hbm, hist, idxb, ridx, wb_in, wb_out, acc,
                sem_d):
    c = lax.axis_index("c")
    s = lax.axis_index("s")
    w = c * 16 + s
    zeros16 = jnp.zeros((16,), jnp.float32)
    ones16 = jnp.ones((16,), jnp.float32)
    iota16 = lax.iota(jnp.int32, 16)

    def _z(i, _):
        hist[i] = zeros16
        return 0
    lax.fori_loop(0, HIST_R, _z, 0)

    def _z2(i, _):
        wb_out[i] = zeros16
        return 0
    lax.fori_loop(0, 784, _z2, 0)
    R_T = HIST_R // 16
    pltpu.sync_copy(wb_out.at[pl.ds(0, R_T)], acc.at[pl.ds(s * R_T, R_T)])
    plsc.subcore_barrier()

    pltpu.async_copy(dst_hbm.at[w, pl.ds(0, 1024)], idxb.at[0], sem_d)

    def _count(k, _):
        p = lax.rem(k, 2)
        pltpu.make_async_copy(dst_hbm.at[w, pl.ds(0, 1024)], idxb.at[p],
                              sem_d).wait()

        @pl.when(k <= CH_D - 2)
        def _():
            pltpu.async_copy(dst_hbm.at[w, pl.ds((k + 1) * 1024, 1024)],
                             idxb.at[lax.rem(k + 1, 2)], sem_d)
        for j in range(64):
            d = idxb[p, pl.ds(j * 16, 16)]
            r = jnp.right_shift(d, 4)
            l = jnp.bitwise_and(d, 15)
            plsc.addupdate_scatter(hist, [r, l], ones16)
        return 0
    lax.fori_loop(0, CH_D, _count, 0)

    def _red(rc, _):
        base = rc * 128
        for j in range(8):
            ridx[pl.ds(j * 16, 16)] = iota16 + (base + j * 16)
        pltpu.sync_copy(hist.at[pl.ds(base, 128)], acc.at[ridx], add=True)
        return 0
    lax.fori_loop(0, HIST_R // 128, _red, 0)
    plsc.subcore_barrier()

    def _wb(k, _):
        r0 = s * R_T + k * 49
        pltpu.sync_copy(acc.at[pl.ds(r0, 49)], wb_in)

        def _rep(j, _):
            rr = jnp.full((16,), jnp.right_shift(j, 4), jnp.int32)
            ll = jnp.full((16,), jnp.bitwise_and(j, 15), jnp.int32)
            wb_out[j] = plsc.load_gather(wb_in, [rr, ll])
            return 0
        lax.fori_loop(0, 784, _rep, 0)
        pltpu.sync_copy(wb_out, out_hbm.at[c, pl.ds(s * 6272 + k * 784, 784)])
        return 0
    lax.fori_loop(0, 8, _wb, 0)


_PASS_SCRATCH = [
    pltpu.VMEM((4, 4, 128), jnp.int32),
    pltpu.VMEM((4, 4, 128), jnp.int32),
    pltpu.VMEM((2, 4, 128, 16), jnp.float32),
    pltpu.VMEM((392, 16), jnp.float32),
    pltpu.VMEM_SHARED((NPAD, 16), jnp.float32),
    pltpu.SemaphoreType.DMA,
    pltpu.SemaphoreType.DMA,
    pltpu.SemaphoreType.DMA,
]


def _edge_pipeline(gref, src_hbm, dst_hbm, acc, idx_s, idx_d, rows,
                   sem_i, sem_g, sem_sc, w):
    NG = EPT // (4 * 128)

    def _issue_idx(g):
        r = lax.rem(g, 4)
        for b in range(4):
            pltpu.async_copy(src_hbm.at[w, pl.ds((g * 4 + b) * 128, 128)],
                             idx_s.at[r, b], sem_i)
            pltpu.async_copy(dst_hbm.at[w, pl.ds((g * 4 + b) * 128, 128)],
                             idx_d.at[r, b], sem_i)

    def _drain_idx(g):
        r = lax.rem(g, 4)
        for b in range(4):
            pltpu.make_async_copy(src_hbm.at[w, pl.ds(0, 128)],
                                  idx_s.at[r, b], sem_i).wait()
            pltpu.make_async_copy(dst_hbm.at[w, pl.ds(0, 128)],
                                  idx_d.at[r, b], sem_i).wait()

    def _issue_gather(g):
        r = lax.rem(g, 4)
        p = lax.rem(g, 2)
        for b in range(4):
            pltpu.async_copy(gref.at[idx_s.at[r, b]], rows.at[p, b], sem_g)

    def _drain_gather(g):
        r = lax.rem(g, 4)
        p = lax.rem(g, 2)
        for b in range(4):
            pltpu.make_async_copy(gref.at[idx_s.at[r, b]], rows.at[p, b],
                                  sem_g).wait()

    def _issue_scatter(g):
        r = lax.rem(g, 4)
        p = lax.rem(g, 2)
        for b in range(4):
            pltpu.async_copy(rows.at[p, b], acc.at[idx_d.at[r, b]], sem_sc,
                             add=True)

    def _drain_scatter(g):
        r = lax.rem(g, 4)
        p = lax.rem(g, 2)
        for b in range(4):
            pltpu.make_async_copy(rows.at[p, b], acc.at[idx_d.at[r, b]],
                                  sem_sc).wait()

    _issue_idx(jnp.int32(0))

    def _pipe(i, _):
        @pl.when(jnp.logical_and(i >= 2, i <= NG + 1))
        def _():
            _drain_scatter(i - 2)

        @pl.when(i <= NG - 1)
        def _():
            _drain_idx(i)

        @pl.when(jnp.logical_and(i >= 1, i <= NG))
        def _():
            _drain_gather(i - 1)
            _issue_scatter(i - 1)

        @pl.when(i <= NG - 1)
        def _():
            _issue_gather(i)

        @pl.when(i <= NG - 2)
        def _():
            _issue_idx(i + 1)
        return 0
    lax.fori_loop(0, NG + 2, _pipe, 0)


def _zero_acc(acc, wb, base_rows):
    zeros16 = jnp.zeros((16,), jnp.float32)

    def _z(i, _):
        wb[i] = zeros16
        return 0
    lax.fori_loop(0, 392, _z, 0)

    def _zc(k, _):
        pltpu.sync_copy(wb, acc.at[pl.ds(base_rows + k * 392, 392)])
        return 0
    lax.fori_loop(0, 16, _zc, 0)


def _write_back(acc, wb, out_hbm, c, base_rows):
    def _wb(k, _):
        r0 = base_rows + k * 392
        pltpu.sync_copy(acc.at[pl.ds(r0, 392)], wb)
        pltpu.sync_copy(wb, out_hbm.at[c, pl.ds(r0, 392)])
        return 0
    lax.fori_loop(0, 16, _wb, 0)


@functools.partial(
    pl.kernel,
    out_type=jax.ShapeDtypeStruct((2, NPAD, 16), jnp.float32),
    mesh=_mesh,
    scratch_types=_PASS_SCRATCH,
    compiler_params=_sc_params,
)
def _pass_kernel(g_hbm, src_hbm, dst_hbm, out_hbm, idx_s, idx_d, rows, wb,
                 acc, sem_i, sem_g, sem_sc):
    c = lax.axis_index("c")
    s = lax.axis_index("s")
    base_rows = s * (NPAD // 16)
    _zero_acc(acc, wb, base_rows)
    plsc.subcore_barrier()
    _edge_pipeline(g_hbm, src_hbm, dst_hbm, acc, idx_s, idx_d, rows,
                   sem_i, sem_g, sem_sc, c * 16 + s)
    plsc.subcore_barrier()
    _write_back(acc, wb, out_hbm, c, base_rows)


@functools.partial(
    pl.kernel,
    out_type=jax.ShapeDtypeStruct((2, NPAD, 16), jnp.float32),
    mesh=_mesh,
    scratch_types=_PASS_SCRATCH,
    compiler_params=_sc_params,
)
def _pass2_kernel(g2s_hbm, src_hbm, dst_hbm, out_hbm, idx_s, idx_d, rows, wb,
                  acc, sem_i, sem_g, sem_sc):
    c = lax.axis_index("c")
    s = lax.axis_index("s")
    base_rows = s * (NPAD // 16)
    _zero_acc(acc, wb, base_rows)
    plsc.subcore_barrier()
    gref = g2s_hbm.at[c]
    _edge_pipeline(gref, src_hbm, dst_hbm, acc, idx_s, idx_d, rows,
                   sem_i, sem_g, sem_sc, 2 * s)
    _edge_pipeline(gref, src_hbm, dst_hbm, acc, idx_s, idx_d, rows,
                   sem_i, sem_g, sem_sc, 2 * s + 1)
    plsc.subcore_barrier()
    _write_back(acc, wb, out_hbm, c, base_rows)


BLK = 6272
GRID = NPAD // BLK


def _tcA_body(x_ref, w1_ref, degp_ref, g1_ref, dis_ref):
    deg = degp_ref[0] + degp_ref[1] + 1.0
    dis = lax.rsqrt(deg)
    h = jnp.dot(x_ref[...], w1_ref[...], preferred_element_type=jnp.float32)
    g1_ref[...] = h * dis
    dis_ref[...] = dis


def _tcA(x_pad, W1, degp):
    return pl.pallas_call(
        _tcA_body,
        grid=(GRID,),
        in_specs=[
            pl.BlockSpec((BLK, 8), lambda i: (i, 0)),
            pl.BlockSpec((8, 16), lambda i: (0, 0)),
            pl.BlockSpec((2, BLK, 16), lambda i: (0, i, 0)),
        ],
        out_specs=[
            pl.BlockSpec((BLK, 16), lambda i: (i, 0)),
            pl.BlockSpec((BLK, 16), lambda i: (i, 0)),
        ],
        out_shape=[
            jax.ShapeDtypeStruct((NPAD, 16), jnp.float32),
            jax.ShapeDtypeStruct((NPAD, 16), jnp.float32),
        ],
    )(x_pad, W1, degp)


def _tcB_body(s1_ref, g1_ref, dis_ref, b1_ref, w2_ref, g2s_ref):
    dis = dis_ref[...]
    o1 = jnp.maximum(dis * (s1_ref[0] + s1_ref[1] + g1_ref[...]) + b1_ref[...],
                     0.0)
    h2 = jnp.dot(o1, w2_ref[...], preferred_element_type=jnp.float32)
    g2s_ref[0] = h2[:, :16] * dis
    g2s_ref[1] = h2[:, 16:] * dis


def _tcB(S1, g1, dis, b1, W2):
    return pl.pallas_call(
        _tcB_body,
        grid=(GRID,),
        in_specs=[
            pl.BlockSpec((2, BLK, 16), lambda i: (0, i, 0)),
            pl.BlockSpec((BLK, 16), lambda i: (i, 0)),
            pl.BlockSpec((BLK, 16), lambda i: (i, 0)),
            pl.BlockSpec((1, 16), lambda i: (0, 0)),
            pl.BlockSpec((16, 32), lambda i: (0, 0)),
        ],
        out_specs=pl.BlockSpec((2, BLK, 16), lambda i: (0, i, 0)),
        out_shape=jax.ShapeDtypeStruct((2, NPAD, 16), jnp.float32),
    )(S1, g1, dis, b1, W2)


def _tcC_body(s2_ref, g2s_ref, dis_ref, b2_ref, out_ref):
    dis = dis_ref[...]
    oa = dis * (s2_ref[0] + g2s_ref[0]) + b2_ref[:, :16]
    ob = dis * (s2_ref[1] + g2s_ref[1]) + b2_ref[:, 16:]
    out_ref[...] = jnp.concatenate([oa, ob], axis=1)


def _tcC(S2, g2s, dis, b2):
    return pl.pallas_call(
        _tcC_body,
        grid=(GRID,),
        in_specs=[
            pl.BlockSpec((2, BLK, 16), lambda i: (0, i, 0)),
            pl.BlockSpec((2, BLK, 16), lambda i: (0, i, 0)),
            pl.BlockSpec((BLK, 16), lambda i: (i, 0)),
            pl.BlockSpec((1, 32), lambda i: (0, 0)),
        ],
        out_specs=pl.BlockSpec((BLK, 32), lambda i: (i, 0)),
        out_shape=jax.ShapeDtypeStruct((NPAD, 32), jnp.float32),
    )(S2, g2s, dis, b2)


def kernel(x_piece, edge_index_piece, W1, b1, W2, b2):
    x_pad = jnp.pad(x_piece, ((0, NPAD - N), (0, 0)))
    n_fake = EPAD - E
    fake = (N + (jnp.arange(n_fake, dtype=jnp.int32) % PAD_NODES))
    src2 = jnp.concatenate([edge_index_piece[0], fake]).reshape(NW, EPT)
    dst2 = jnp.concatenate([edge_index_piece[1], fake]).reshape(NW, EPT)

    degp = _deg_kernel(dst2)
    g1, dis = _tcA(x_pad, W1, degp)
    S1 = _pass_kernel(g1, src2, dst2)
    g2s = _tcB(S1, g1, dis, b1.reshape(1, 16), W2)
    S2 = _pass2_kernel(g2s, src2, dst2)
    out = _tcC(S2, g2s, dis, b2.reshape(1, 32))
    return out[:N]


# speedup vs baseline: 81.5350x; 1.1926x over previous
import functools

import jax
import jax.numpy as jnp
from jax import lax
from jax.experimental import pallas as pl
from jax.experimental.pallas import tpu as pltpu
from jax.experimental.pallas import tpu_sc as plsc

N = 100000
E = 6400000
NPAD = 100352
HIST_R = NPAD // 16
NW = 32
EPT = 200704
EPAD = NW * EPT
CH = EPT // 128
CH_D = EPT // 1024
PAD_NODES = NPAD - N

_mesh = plsc.VectorSubcoreMesh(core_axis_name="c", subcore_axis_name="s")
_sc_params = pltpu.CompilerParams(
    needs_layout_passes=False, use_tc_tiling_on_sc=False)


@functools.partial(
    pl.kernel,
    out_type=jax.ShapeDtypeStruct((2, NPAD, 16), jnp.float32),
    mesh=_mesh,
    scratch_types=[
        pltpu.VMEM((HIST_R, 16), jnp.float32),
        pltpu.VMEM((2, 1024), jnp.int32),
        pltpu.VMEM((128,), jnp.int32),
        pltpu.VMEM((49, 16), jnp.float32),
        pltpu.VMEM((784, 16), jnp.float32),
        pltpu.VMEM_SHARED((HIST_R, 16), jnp.float32),
        pltpu.SemaphoreType.DMA,
    ],
    compiler_params=_sc_params,
)
def _deg_kernel(dst_hbm, out_hbm, hist, idxb, ridx, wb_in, wb_out, acc,
                sem_d):
    c = lax.axis_index("c")
    s = lax.axis_index("s")
    w = c * 16 + s
    zeros16 = jnp.zeros((16,), jnp.float32)
    ones16 = jnp.ones((16,), jnp.float32)
    iota16 = lax.iota(jnp.int32, 16)

    def _z(i, _):
        hist[i] = zeros16
        return 0
    lax.fori_loop(0, HIST_R, _z, 0)

    def _z2(i, _):
        wb_out[i] = zeros16
        return 0
    lax.fori_loop(0, 784, _z2, 0)
    R_T = HIST_R // 16
    pltpu.sync_copy(wb_out.at[pl.ds(0, R_T)], acc.at[pl.ds(s * R_T, R_T)])
    plsc.subcore_barrier()

    pltpu.async_copy(dst_hbm.at[w, pl.ds(0, 1024)], idxb.at[0], sem_d)

    def _count(k, _):
        p = lax.rem(k, 2)
        pltpu.make_async_copy(dst_hbm.at[w, pl.ds(0, 1024)], idxb.at[p],
                              sem_d).wait()

        @pl.when(k <= CH_D - 2)
        def _():
            pltpu.async_copy(dst_hbm.at[w, pl.ds((k + 1) * 1024, 1024)],
                             idxb.at[lax.rem(k + 1, 2)], sem_d)
        for j in range(64):
            d = idxb[p, pl.ds(j * 16, 16)]
            r = jnp.right_shift(d, 4)
            l = jnp.bitwise_and(d, 15)
            plsc.addupdate_scatter(hist, [r, l], ones16)
        return 0
    lax.fori_loop(0, CH_D, _count, 0)

    def _red(rc, _):
        base = rc * 128
        for j in range(8):
            ridx[pl.ds(j * 16, 16)] = iota16 + (base + j * 16)
        pltpu.sync_copy(hist.at[pl.ds(base, 128)], acc.at[ridx], add=True)
        return 0
    lax.fori_loop(0, HIST_R // 128, _red, 0)
    plsc.subcore_barrier()

    def _wb(k, _):
        r0 = s * R_T + k * 49
        pltpu.sync_copy(acc.at[pl.ds(r0, 49)], wb_in)

        def _rep(j, _):
            rr = jnp.full((16,), jnp.right_shift(j, 4), jnp.int32)
            ll = jnp.full((16,), jnp.bitwise_and(j, 15), jnp.int32)
            wb_out[j] = plsc.load_gather(wb_in, [rr, ll])
            return 0
        lax.fori_loop(0, 784, _rep, 0)
        pltpu.sync_copy(wb_out, out_hbm.at[c, pl.ds(s * 6272 + k * 784, 784)])
        return 0
    lax.fori_loop(0, 8, _wb, 0)


_PASS_SCRATCH = [
    pltpu.VMEM((4, 4, 128), jnp.int32),
    pltpu.VMEM((4, 4, 128), jnp.int32),
    pltpu.VMEM((2, 4, 128, 16), jnp.float32),
    pltpu.VMEM((392, 16), jnp.float32),
    pltpu.VMEM_SHARED((NPAD, 16), jnp.float32),
    pltpu.SemaphoreType.DMA,
    pltpu.SemaphoreType.DMA,
    pltpu.SemaphoreType.DMA,
]


def _edge_pipeline(gref, src_hbm, dst_hbm, acc, idx_s, idx_d, rows,
                   sem_i, sem_g, sem_sc, w):
    NG = EPT // (4 * 128)

    def _issue_idx(g):
        r = lax.rem(g, 4)
        for b in range(4):
            pltpu.async_copy(src_hbm.at[w, pl.ds((g * 4 + b) * 128, 128)],
                             idx_s.at[r, b], sem_i)
            pltpu.async_copy(dst_hbm.at[w, pl.ds((g * 4 + b) * 128, 128)],
                             idx_d.at[r, b], sem_i)

    def _drain_idx(g):
        r = lax.rem(g, 4)
        for b in range(4):
            pltpu.make_async_copy(src_hbm.at[w, pl.ds(0, 128)],
                                  idx_s.at[r, b], sem_i).wait()
            pltpu.make_async_copy(dst_hbm.at[w, pl.ds(0, 128)],
                                  idx_d.at[r, b], sem_i).wait()

    def _issue_gather(g):
        r = lax.rem(g, 4)
        p = lax.rem(g, 2)
        for b in range(4):
            pltpu.async_copy(gref.at[idx_s.at[r, b]], rows.at[p, b], sem_g)

    def _drain_gather(g):
        r = lax.rem(g, 4)
        p = lax.rem(g, 2)
        for b in range(4):
            pltpu.make_async_copy(gref.at[idx_s.at[r, b]], rows.at[p, b],
                                  sem_g).wait()

    def _issue_scatter(g):
        r = lax.rem(g, 4)
        p = lax.rem(g, 2)
        for b in range(4):
            pltpu.async_copy(rows.at[p, b], acc.at[idx_d.at[r, b]], sem_sc,
                             add=True)

    def _drain_scatter(g):
        r = lax.rem(g, 4)
        p = lax.rem(g, 2)
        for b in range(4):
            pltpu.make_async_copy(rows.at[p, b], acc.at[idx_d.at[r, b]],
                                  sem_sc).wait()

    _issue_idx(jnp.int32(0))

    def _pipe(i, _):
        @pl.when(jnp.logical_and(i >= 2, i <= NG + 1))
        def _():
            _drain_scatter(i - 2)

        @pl.when(i <= NG - 1)
        def _():
            _drain_idx(i)

        @pl.when(jnp.logical_and(i >= 1, i <= NG))
        def _():
            _drain_gather(i - 1)
            _issue_scatter(i - 1)

        @pl.when(i <= NG - 1)
        def _():
            _issue_gather(i)

        @pl.when(i <= NG - 2)
        def _():
            _issue_idx(i + 1)
        return 0
    lax.fori_loop(0, NG + 2, _pipe, 0)


def _zero_acc(acc, wb, base_rows):
    zeros16 = jnp.zeros((16,), jnp.float32)

    def _z(i, _):
        wb[i] = zeros16
        return 0
    lax.fori_loop(0, 392, _z, 0)

    def _zc(k, _):
        pltpu.sync_copy(wb, acc.at[pl.ds(base_rows + k * 392, 392)])
        return 0
    lax.fori_loop(0, 16, _zc, 0)


def _write_back(acc, wb, out_hbm, c, base_rows):
    def _wb(k, _):
        r0 = base_rows + k * 392
        pltpu.sync_copy(acc.at[pl.ds(r0, 392)], wb)
        pltpu.sync_copy(wb, out_hbm.at[c, pl.ds(r0, 392)])
        return 0
    lax.fori_loop(0, 16, _wb, 0)


@functools.partial(
    pl.kernel,
    out_type=jax.ShapeDtypeStruct((2, NPAD, 16), jnp.float32),
    mesh=_mesh,
    scratch_types=_PASS_SCRATCH,
    compiler_params=_sc_params,
)
def _pass_kernel(g_hbm, src_hbm, dst_hbm, out_hbm, idx_s, idx_d, rows, wb,
                 acc, sem_i, sem_g, sem_sc):
    c = lax.axis_index("c")
    s = lax.axis_index("s")
    base_rows = s * (NPAD // 16)
    _zero_acc(acc, wb, base_rows)
    plsc.subcore_barrier()
    _edge_pipeline(g_hbm, src_hbm, dst_hbm, acc, idx_s, idx_d, rows,
                   sem_i, sem_g, sem_sc, c * 16 + s)
    plsc.subcore_barrier()
    _write_back(acc, wb, out_hbm, c, base_rows)


@functools.partial(
    pl.kernel,
    out_type=jax.ShapeDtypeStruct((2, NPAD, 16), jnp.float32),
    mesh=_mesh,
    scratch_types=_PASS_SCRATCH,
    compiler_params=_sc_params,
)
def _pass2_kernel(g2s_hbm, src_hbm, dst_hbm, out_hbm, idx_s, idx_d, rows, wb,
                  acc, sem_i, sem_g, sem_sc):
    c = lax.axis_index("c")
    s = lax.axis_index("s")
    base_rows = s * (NPAD // 16)
    _zero_acc(acc, wb, base_rows)
    plsc.subcore_barrier()
    gref = g2s_hbm.at[c]
    _edge_pipeline(gref, src_hbm, dst_hbm, acc, idx_s, idx_d, rows,
                   sem_i, sem_g, sem_sc, 2 * s)
    _edge_pipeline(gref, src_hbm, dst_hbm, acc, idx_s, idx_d, rows,
                   sem_i, sem_g, sem_sc, 2 * s + 1)
    plsc.subcore_barrier()
    _write_back(acc, wb, out_hbm, c, base_rows)


RFLAT = NPAD // 16
BLKR = RFLAT // 4
GRID = 4


def _tcA_body(x_ref, b1k_ref, degp_ref, g1_ref, dis_ref):
    deg = degp_ref[0] + degp_ref[1] + 1.0
    dis = lax.rsqrt(deg)
    h = jnp.dot(x_ref[...], b1k_ref[...], preferred_element_type=jnp.float32)
    g1_ref[...] = h * dis
    dis_ref[...] = dis


def _tcA(x128, B1k, degp2):
    return pl.pallas_call(
        _tcA_body,
        grid=(GRID,),
        in_specs=[
            pl.BlockSpec((BLKR, 128), lambda i: (i, 0)),
            pl.BlockSpec((128, 256), lambda i: (0, 0)),
            pl.BlockSpec((2, BLKR, 256), lambda i: (0, i, 0)),
        ],
        out_specs=[
            pl.BlockSpec((BLKR, 256), lambda i: (i, 0)),
            pl.BlockSpec((BLKR, 256), lambda i: (i, 0)),
        ],
        out_shape=[
            jax.ShapeDtypeStruct((RFLAT, 256), jnp.float32),
            jax.ShapeDtypeStruct((RFLAT, 256), jnp.float32),
        ],
    )(x128, B1k, degp2)


def _tcB_body(s1_ref, g1_ref, dis_ref, b1t_ref, b2lo_ref, b2hi_ref, g2s_ref):
    dis = dis_ref[...]
    o1 = jnp.maximum(
        dis * (s1_ref[0] + s1_ref[1] + g1_ref[...]) + b1t_ref[...], 0.0)
    g2s_ref[0] = jnp.dot(o1, b2lo_ref[...],
                         preferred_element_type=jnp.float32) * dis
    g2s_ref[1] = jnp.dot(o1, b2hi_ref[...],
                         preferred_element_type=jnp.float32) * dis


def _tcB(S1, g1, dis, b1t, B2lo, B2hi):
    return pl.pallas_call(
        _tcB_body,
        grid=(GRID,),
        in_specs=[
            pl.BlockSpec((2, BLKR, 256), lambda i: (0, i, 0)),
            pl.BlockSpec((BLKR, 256), lambda i: (i, 0)),
            pl.BlockSpec((BLKR, 256), lambda i: (i, 0)),
            pl.BlockSpec((1, 256), lambda i: (0, 0)),
            pl.BlockSpec((256, 256), lambda i: (0, 0)),
            pl.BlockSpec((256, 256), lambda i: (0, 0)),
        ],
        out_specs=pl.BlockSpec((2, BLKR, 256), lambda i: (0, i, 0)),
        out_shape=jax.ShapeDtypeStruct((2, RFLAT, 256), jnp.float32),
    )(S1, g1, dis, b1t, B2lo, B2hi)


def _tcC_body(s2_ref, g2s_ref, dis_ref, b2lo_ref, b2hi_ref, lo_ref, hi_ref):
    dis = dis_ref[...]
    lo_ref[...] = dis * (s2_ref[0] + g2s_ref[0]) + b2lo_ref[...]
    hi_ref[...] = dis * (s2_ref[1] + g2s_ref[1]) + b2hi_ref[...]


def _tcC(S2, g2s, dis, b2lot, b2hit):
    return pl.pallas_call(
        _tcC_body,
        grid=(GRID,),
        in_specs=[
            pl.BlockSpec((2, BLKR, 256), lambda i: (0, i, 0)),
            pl.BlockSpec((2, BLKR, 256), lambda i: (0, i, 0)),
            pl.BlockSpec((BLKR, 256), lambda i: (i, 0)),
            pl.BlockSpec((1, 256), lambda i: (0, 0)),
            pl.BlockSpec((1, 256), lambda i: (0, 0)),
        ],
        out_specs=[
            pl.BlockSpec((BLKR, 256), lambda i: (i, 0)),
            pl.BlockSpec((BLKR, 256), lambda i: (i, 0)),
        ],
        out_shape=[
            jax.ShapeDtypeStruct((RFLAT, 256), jnp.float32),
            jax.ShapeDtypeStruct((RFLAT, 256), jnp.float32),
        ],
    )(S2, g2s, dis, b2lot, b2hit)


def kernel(x_piece, edge_index_piece, W1, b1, W2, b2):
    x_pad = jnp.pad(x_piece, ((0, NPAD - N), (0, 0)))
    n_fake = EPAD - E
    fake = (N + (jnp.arange(n_fake, dtype=jnp.int32) % PAD_NODES))
    src2 = jnp.concatenate([edge_index_piece[0], fake]).reshape(NW, EPT)
    dst2 = jnp.concatenate([edge_index_piece[1], fake]).reshape(NW, EPT)

    eye16 = jnp.eye(16, dtype=jnp.float32)
    B1k = jnp.kron(eye16, W1)
    B2lo = jnp.kron(eye16, W2[:, :16])
    B2hi = jnp.kron(eye16, W2[:, 16:])
    b1t = jnp.tile(b1, 16).reshape(1, 256)
    b2lot = jnp.tile(b2[:16], 16).reshape(1, 256)
    b2hit = jnp.tile(b2[16:], 16).reshape(1, 256)
    x128 = x_pad.reshape(RFLAT, 128)

    degp = _deg_kernel(dst2)
    g1, dis = _tcA(x128, B1k, degp.reshape(2, RFLAT, 256))
    S1 = _pass_kernel(g1.reshape(NPAD, 16), src2, dst2)
    g2s = _tcB(S1.reshape(2, RFLAT, 256), g1, dis, b1t, B2lo, B2hi)
    S2 = _pass2_kernel(g2s.reshape(2, NPAD, 16), src2, dst2)
    lo, hi = _tcC(S2.reshape(2, RFLAT, 256), g2s, dis, b2lot, b2hit)
    out = jnp.concatenate([lo.reshape(NPAD, 16), hi.reshape(NPAD, 16)],
                          axis=1)
    return out[:N]
